# Initial kernel scaffold; baseline (speedup 1.0000x reference)
import jax
import jax.numpy as jnp
from jax.experimental import pallas as pl


def kernel(x, table):
    raise NotImplementedError("write your pallas kernel here")


# speedup vs baseline: 1.5476x; 1.5476x over previous
import jax
import jax.numpy as jnp
from jax import lax
from jax.experimental import pallas as pl
from jax.experimental.pallas import tpu as pltpu
from jax.experimental.pallas import tpu_sc as plsc

NC = 2
NS = 16
NW = NC * NS
G = 128
SG = 8


def _body(x_hbm, table_hbm, out_hbm, idx_v, rows_v, sem):
    wid = lax.axis_index("s") * NC + lax.axis_index("c")
    gw = x_hbm.shape[0] // NW
    nsup = gw // SG

    @pl.loop(0, nsup)
    def _chunk(s):
        g0 = wid * gw + s * SG
        pltpu.sync_copy(x_hbm.at[pl.ds(g0, SG)], idx_v)
        copies = [
            pltpu.async_copy(table_hbm.at[idx_v.at[j]], rows_v.at[j], sem)
            for j in range(SG)
        ]
        for c in copies:
            c.wait()
        pltpu.sync_copy(rows_v, out_hbm.at[pl.ds(g0, SG)])


def kernel(x, table):
    B, F = x.shape
    V, D = table.shape
    n = B * F
    x_flat = x.reshape(n // G, G).astype(jnp.int32)
    mesh = plsc.VectorSubcoreMesh(core_axis_name="c", subcore_axis_name="s")
    out = pl.kernel(
        _body,
        out_type=jax.ShapeDtypeStruct((n // G, G, D), jnp.float32),
        mesh=mesh,
        scratch_types=[
            pltpu.VMEM((SG, G), jnp.int32),
            pltpu.VMEM((SG, G, D), jnp.float32),
            pltpu.SemaphoreType.DMA,
        ],
        compiler_params=pltpu.CompilerParams(use_tc_tiling_on_sc=False),
    )(x_flat, table)
    return out.reshape(B, F, D)


# speedup vs baseline: 1.5706x; 1.0148x over previous
import jax
import jax.numpy as jnp
from jax import lax
from jax.experimental import pallas as pl
from jax.experimental.pallas import tpu as pltpu
from jax.experimental.pallas import tpu_sc as plsc

NC = 2
NS = 16
NW = NC * NS
G = 128
SG = 13


def _body(x_hbm, table_hbm, out_hbm, idx_v, rows_v, sem0, sem1):
    wid = lax.axis_index("s") * NC + lax.axis_index("c")
    gw = x_hbm.shape[0] // NW
    nsup = gw // SG
    g_base = wid * gw
    sems = (sem0, sem1)

    def load_and_fire(b, chunk):
        g0 = g_base + chunk * SG
        pltpu.sync_copy(x_hbm.at[pl.ds(g0, SG)], idx_v.at[b])
        for j in range(SG):
            pltpu.async_copy(table_hbm.at[idx_v.at[b, j]], rows_v.at[b, j],
                             sems[b])

    def drain_and_store(b, chunk):
        g0 = g_base + chunk * SG
        for j in range(SG):
            pltpu.make_async_copy(table_hbm.at[idx_v.at[b, j]],
                                  rows_v.at[b, j], sems[b]).wait()
        pltpu.sync_copy(rows_v.at[b], out_hbm.at[pl.ds(g0, SG)])

    load_and_fire(0, 0)

    @pl.loop(0, nsup, step=2)
    def _chunks(s):
        load_and_fire(1, s + 1)
        drain_and_store(0, s)

        @pl.when(s + 2 < nsup)
        def _():
            load_and_fire(0, s + 2)

        drain_and_store(1, s + 1)


def kernel(x, table):
    B, F = x.shape
    V, D = table.shape
    n = B * F
    x_flat = x.reshape(n // G, G).astype(jnp.int32)
    mesh = plsc.VectorSubcoreMesh(core_axis_name="c", subcore_axis_name="s")
    out = pl.kernel(
        _body,
        out_type=jax.ShapeDtypeStruct((n // G, G, D), jnp.float32),
        mesh=mesh,
        scratch_types=[
            pltpu.VMEM((2, SG, G), jnp.int32),
            pltpu.VMEM((2, SG, G, D), jnp.float32),
            pltpu.SemaphoreType.DMA,
            pltpu.SemaphoreType.DMA,
        ],
        compiler_params=pltpu.CompilerParams(use_tc_tiling_on_sc=False),
    )(x_flat, table)
    return out.reshape(B, F, D)
